# baseline (speedup 1.0000x reference)
import functools

import jax
import jax.numpy as jnp
from jax import lax
from jax.experimental import pallas as pl
from jax.experimental.pallas import tpu as pltpu
from jax.experimental.pallas import tpu_sc as plsc

NUM_CLASSES = 1000000
EMBED_DIM = 64
BATCH = 16384

NC = 2
NS = 16
NW = NC * NS
B_PER_W = BATCH // NW
NBUF = 8


def _extract_row(slice_ref, lane, rows_ref, row_off):
    lane_vec = jnp.full((16,), lane, dtype=jnp.int32)
    for j in range(EMBED_DIM // 16):
        c_vec = lax.iota(jnp.int32, 16) + 16 * j
        vals = plsc.load_gather(slice_ref, [c_vec, lane_vec])
        rows_ref[pl.ds(row_off + 16 * j, 16)] = vals


NBLK = (NUM_CLASSES + 127) // 128
MAX_T = (NBLK + NW - 1) // NW
Q_MAX = (NBLK - 1) * 128
NBUF2 = 4
ROW_RING = 8


def _lane_select(vec, lane):
    return jnp.sum(jnp.where(lax.iota(jnp.int32, 16) == lane, vec, 0))


def _sc_gather(table_t, class_index):
    mesh = plsc.VectorSubcoreMesh(core_axis_name="c", subcore_axis_name="s")

    @functools.partial(
        pl.kernel,
        mesh=mesh,
        out_type=jax.ShapeDtypeStruct((BATCH * EMBED_DIM,), jnp.float32),
        scratch_types=[
            pltpu.VMEM((BATCH,), jnp.int32),
            pltpu.VMEM((BATCH + 16,), jnp.int32),
            pltpu.VMEM((BATCH + 16,), jnp.int32),
            pltpu.VMEM((256,), jnp.int32),
            pltpu.VMEM((272,), jnp.int32),
            [pltpu.VMEM((EMBED_DIM, 128), jnp.float32) for _ in range(NBUF2)],
            pltpu.VMEM((ROW_RING * EMBED_DIM,), jnp.float32),
            [pltpu.SemaphoreType.DMA for _ in range(NBUF2)],
            pltpu.SemaphoreType.DMA,
        ],
        compiler_params=pltpu.CompilerParams(needs_layout_passes=False),
    )
    def gather(
        table_hbm, idx_hbm, out_hbm,
        idx_all, my_r, my_pos, pres, blist, bufs, rowring, sems, sem_row,
    ):
        wid = lax.axis_index("s") * NC + lax.axis_index("c")
        iota = lax.iota(jnp.int32, 16)
        pltpu.sync_copy(idx_hbm, idx_all)
        zeros = jnp.zeros((16,), jnp.int32)
        for t in range(16):
            pres[pl.ds(16 * t, 16)] = zeros

        def filt(g, cnt):
            rv = idx_all[pl.ds(pl.multiple_of(16 * g, 16), 16)]
            blk = lax.shift_right_logical(rv, 7)
            mine = (blk & (NW - 1)) == wid
            plsc.store_compressed(my_r.at[pl.ds(cnt, 16)], rv, mask=mine)
            plsc.store_compressed(my_pos.at[pl.ds(cnt, 16)], 16 * g + iota, mask=mine)
            tv = lax.shift_right_logical(blk, 5)
            plsc.store_scatter(pres, [tv], zeros + 1, mask=mine)
            return cnt + jnp.max(plsc.all_reduce_population_count(mine))

        cnt = lax.fori_loop(0, BATCH // 16, filt, 0)

        def compact(t16, bcnt):
            pv = pres[pl.ds(16 * t16, 16)]
            m = pv > 0
            blk_vec = NW * (16 * t16 + iota) + wid
            plsc.store_compressed(blist.at[pl.ds(bcnt, 16)], blk_vec, mask=m)
            return bcnt + jnp.max(plsc.all_reduce_population_count(m))

        bcnt = lax.fori_loop(0, 16, compact, 0)

        def read_blist(i):
            grp = pl.multiple_of(
                lax.shift_left(lax.shift_right_logical(i, 4), 4), 16
            )
            return _lane_select(blist[pl.ds(grp, 16)], i & 15)

        def fire(i, b):
            blk = read_blist(jnp.minimum(i, jnp.maximum(bcnt - 1, 0)))
            q = jnp.clip(lax.shift_left(blk, 7), 0, Q_MAX)
            q = pl.multiple_of(q, 128)
            pltpu.async_copy(table_hbm.at[:, pl.ds(q, 128)], bufs[b], sems[b])

        def drain(b):
            pltpu.make_async_copy(
                table_hbm.at[:, pl.ds(0, 128)], bufs[b], sems[b]
            ).wait()

        def drain_row():
            pltpu.make_async_copy(
                rowring.at[pl.ds(0, EMBED_DIM)],
                out_hbm.at[pl.ds(0, EMBED_DIM)],
                sem_row,
            ).wait()

        for b in range(NBUF2):
            fire(b, b)

        def proc_group(k, f):
            for b in range(NBUF2):
                i = jnp.minimum(NBUF2 * k + b, jnp.maximum(bcnt - 1, 0))
                blk = read_blist(i)
                drain(b)

                def scan(g, f, blk=blk, buf=bufs[b]):
                    gbase = pl.multiple_of(16 * g, 16)
                    rv = my_r[pl.ds(gbase, 16)]
                    posv = my_pos[pl.ds(gbase, 16)]
                    m = (lax.shift_right_logical(rv, 7) == blk) & (
                        gbase + iota < cnt
                    )

                    def ext_cond(carry):
                        m, _ = carry
                        return jnp.any(m)

                    def ext(carry):
                        m, f = carry
                        lane = jnp.max(plsc.all_reduce_ffs(m))
                        r = _lane_select(rv, lane)
                        pos = _lane_select(posv, lane)
                        slot = (f & (ROW_RING - 1)) * EMBED_DIM

                        @pl.when(f >= ROW_RING)
                        def _():
                            drain_row()

                        _extract_row(buf, r & 127, rowring, slot)
                        pltpu.async_copy(
                            rowring.at[pl.ds(slot, EMBED_DIM)],
                            out_hbm.at[pl.ds(pos * EMBED_DIM, EMBED_DIM)],
                            sem_row,
                        )
                        return m & (iota != lane), f + 1

                    m, f = lax.while_loop(ext_cond, ext, (m, f))
                    return f

                f = lax.fori_loop(
                    0, lax.shift_right_logical(cnt + 15, 4), scan, f
                )
                fire(i + NBUF2, b)
            return f

        f = lax.fori_loop(0, lax.shift_right_logical(bcnt + NBUF2 - 1, 2), proc_group, 0)
        for b in range(NBUF2):
            drain(b)
        lax.fori_loop(0, jnp.minimum(f, ROW_RING), lambda i, c: (drain_row(), c)[1], 0)

    return gather(table_t, class_index)


def _mlp_body(emb_ref, w1_ref, b1_ref, w2_ref, b2_ref, out_ref):
    x = emb_ref[...]
    h = jnp.dot(x, w1_ref[...], preferred_element_type=jnp.float32) + b1_ref[...]
    h = h * jax.nn.sigmoid(h)
    out_ref[...] = (
        jnp.dot(h, w2_ref[...], preferred_element_type=jnp.float32) + b2_ref[...]
    )


def _tc_mlp(emb, W1, b1, W2, b2):
    blk = 2048
    grid = (BATCH // blk,)
    return pl.pallas_call(
        _mlp_body,
        grid=grid,
        in_specs=[
            pl.BlockSpec((blk, EMBED_DIM), lambda i: (i, 0)),
            pl.BlockSpec((EMBED_DIM, EMBED_DIM), lambda i: (0, 0)),
            pl.BlockSpec((1, EMBED_DIM), lambda i: (0, 0)),
            pl.BlockSpec((EMBED_DIM, EMBED_DIM), lambda i: (0, 0)),
            pl.BlockSpec((1, EMBED_DIM), lambda i: (0, 0)),
        ],
        out_specs=pl.BlockSpec((blk, EMBED_DIM), lambda i: (i, 0)),
        out_shape=jax.ShapeDtypeStruct((BATCH, EMBED_DIM), jnp.float32),
        compiler_params=pltpu.CompilerParams(
            dimension_semantics=("parallel",),
        ),
    )(emb, W1, b1.reshape(1, EMBED_DIM), W2, b2.reshape(1, EMBED_DIM))


def kernel(class_index, table, W1, b1, W2, b2):
    emb_flat = _sc_gather(table.T, class_index.astype(jnp.int32))
    emb = emb_flat.reshape(BATCH, EMBED_DIM)
    return _tc_mlp(emb, W1, b1, W2, b2)


# speedup vs baseline: 1.4223x; 1.4223x over previous
import functools

import jax
import jax.numpy as jnp
from jax import lax
from jax.experimental import pallas as pl
from jax.experimental.pallas import tpu as pltpu
from jax.experimental.pallas import tpu_sc as plsc

NUM_CLASSES = 1000000
EMBED_DIM = 64
BATCH = 16384

NC = 2
NS = 16
NW = NC * NS
B_PER_W = BATCH // NW
NBUF = 8


def _extract_row(slice_ref, lane, rows_ref, row_off):
    lane_vec = jnp.full((16,), lane, dtype=jnp.int32)
    for j in range(EMBED_DIM // 16):
        c_vec = lax.iota(jnp.int32, 16) + 16 * j
        vals = plsc.load_gather(slice_ref, [c_vec, lane_vec])
        rows_ref[pl.ds(row_off + 16 * j, 16)] = vals


def _sc_gather(table_t, class_index, n):
    mesh = plsc.VectorSubcoreMesh(core_axis_name="c", subcore_axis_name="s")
    b_per_w = n // NW

    @functools.partial(
        pl.kernel,
        mesh=mesh,
        out_type=jax.ShapeDtypeStruct((n * EMBED_DIM,), jnp.float32),
        scratch_types=[
            pltpu.VMEM((b_per_w,), jnp.int32),
            [pltpu.VMEM((EMBED_DIM, 128), jnp.float32) for _ in range(NBUF)],
            pltpu.VMEM((b_per_w * EMBED_DIM,), jnp.float32),
            [pltpu.SemaphoreType.DMA for _ in range(NBUF)],
        ],
        compiler_params=pltpu.CompilerParams(needs_layout_passes=False),
    )
    def gather(table_hbm, idx_hbm, out_hbm, idx_v, bufs, rows_v, sems):
        wid = lax.axis_index("s") * NC + lax.axis_index("c")
        base = wid * b_per_w
        pltpu.sync_copy(idx_hbm.at[pl.ds(base, b_per_w)], idx_v)

        def read_idx(j):
            grp = pl.multiple_of(
                lax.shift_left(lax.shift_right_logical(j, 4), 4), 16
            )
            vec = idx_v[pl.ds(grp, 16)]
            lane = lax.iota(jnp.int32, 16)
            sel = jnp.where(lane == (j & 15), vec, 0)
            return jnp.sum(sel)

        def fire(j, b):
            r = read_idx(j)
            q = pl.multiple_of(lax.shift_left(lax.shift_right_logical(r, 7), 7), 128)
            pltpu.async_copy(table_hbm.at[:, pl.ds(q, 128)], bufs[b], sems[b])

        def drain(b):
            pltpu.make_async_copy(
                table_hbm.at[:, pl.ds(0, 128)], bufs[b], sems[b]
            ).wait()

        for b in range(NBUF):
            fire(b, b)

        def body(k, carry):
            for b in range(NBUF):
                j = jnp.minimum(NBUF * k + b, b_per_w - 1)
                drain(b)
                _extract_row(bufs[b], read_idx(j) & 127, rows_v, j * EMBED_DIM)
                fire(jnp.minimum(j + NBUF, b_per_w - 1), b)
            return carry

        lax.fori_loop(0, (b_per_w + NBUF - 1) // NBUF, body, 0)
        for b in range(NBUF):
            drain(b)
        pltpu.sync_copy(rows_v, out_hbm.at[pl.ds(base * EMBED_DIM, b_per_w * EMBED_DIM)])

    return gather(table_t, class_index)


def _mlp_body(emb_ref, w1_ref, b1_ref, w2_ref, b2_ref, out_ref):
    x = emb_ref[...]
    h = jnp.dot(x, w1_ref[...], preferred_element_type=jnp.float32) + b1_ref[...]
    h = h * jax.nn.sigmoid(h)
    out_ref[...] = (
        jnp.dot(h, w2_ref[...], preferred_element_type=jnp.float32) + b2_ref[...]
    )


def _tc_mlp(emb, W1, b1, W2, b2):
    n = emb.shape[0]
    blk = 2048
    grid = (n // blk,)
    return pl.pallas_call(
        _mlp_body,
        grid=grid,
        in_specs=[
            pl.BlockSpec((blk, EMBED_DIM), lambda i: (i, 0)),
            pl.BlockSpec((EMBED_DIM, EMBED_DIM), lambda i: (0, 0)),
            pl.BlockSpec((1, EMBED_DIM), lambda i: (0, 0)),
            pl.BlockSpec((EMBED_DIM, EMBED_DIM), lambda i: (0, 0)),
            pl.BlockSpec((1, EMBED_DIM), lambda i: (0, 0)),
        ],
        out_specs=pl.BlockSpec((blk, EMBED_DIM), lambda i: (i, 0)),
        out_shape=jax.ShapeDtypeStruct((n, EMBED_DIM), jnp.float32),
        compiler_params=pltpu.CompilerParams(
            dimension_semantics=("parallel",),
        ),
    )(emb, W1, b1.reshape(1, EMBED_DIM), W2, b2.reshape(1, EMBED_DIM))


def kernel(class_index, table, W1, b1, W2, b2):
    idx = class_index.astype(jnp.int32)
    table_t = table.T
    half = BATCH // 2
    outs = []
    for h in range(2):
        emb_flat = _sc_gather(table_t, lax.slice(idx, (h * half,), ((h + 1) * half,)), half)
        outs.append(_tc_mlp(emb_flat.reshape(half, EMBED_DIM), W1, b1, W2, b2))
    return lax.concatenate(outs, 0)


# speedup vs baseline: 1.5420x; 1.0842x over previous
import functools

import jax
import jax.numpy as jnp
from jax import lax
from jax.experimental import pallas as pl
from jax.experimental.pallas import tpu as pltpu
from jax.experimental.pallas import tpu_sc as plsc

NUM_CLASSES = 1000000
EMBED_DIM = 64
BATCH = 16384

NC = 2
NS = 16
NW = NC * NS
B_PER_W = BATCH // NW
NBUF = 8


def _extract_row(slice_ref, lane, rows_ref, row_off):
    lane_vec = jnp.full((16,), lane, dtype=jnp.int32)
    for j in range(EMBED_DIM // 16):
        c_vec = lax.iota(jnp.int32, 16) + 16 * j
        vals = plsc.load_gather(slice_ref, [c_vec, lane_vec])
        rows_ref[pl.ds(row_off + 16 * j, 16)] = vals


def _sc_gather(table_t, class_index):
    mesh = plsc.VectorSubcoreMesh(core_axis_name="c", subcore_axis_name="s")

    @functools.partial(
        pl.kernel,
        mesh=mesh,
        out_type=jax.ShapeDtypeStruct((BATCH * EMBED_DIM,), jnp.float32),
        scratch_types=[
            pltpu.VMEM((B_PER_W,), jnp.int32),
            [pltpu.VMEM((EMBED_DIM, 128), jnp.float32) for _ in range(NBUF)],
            pltpu.VMEM((B_PER_W * EMBED_DIM,), jnp.float32),
            [pltpu.SemaphoreType.DMA for _ in range(NBUF)],
        ],
        compiler_params=pltpu.CompilerParams(needs_layout_passes=False),
    )
    def gather(table_hbm, idx_hbm, out_hbm, idx_v, bufs, rows_v, sems):
        wid = lax.axis_index("s") * NC + lax.axis_index("c")
        base = wid * B_PER_W
        pltpu.sync_copy(idx_hbm.at[pl.ds(base, B_PER_W)], idx_v)

        def read_idx(j):
            grp = pl.multiple_of(
                lax.shift_left(lax.shift_right_logical(j, 4), 4), 16
            )
            vec = idx_v[pl.ds(grp, 16)]
            lane = lax.iota(jnp.int32, 16)
            sel = jnp.where(lane == (j & 15), vec, 0)
            return jnp.sum(sel)

        def fire(j, b):
            r = read_idx(j)
            q = pl.multiple_of(lax.shift_left(lax.shift_right_logical(r, 7), 7), 128)
            pltpu.async_copy(table_hbm.at[:, pl.ds(q, 128)], bufs[b], sems[b])

        def drain(b):
            pltpu.make_async_copy(
                table_hbm.at[:, pl.ds(0, 128)], bufs[b], sems[b]
            ).wait()

        for b in range(NBUF):
            fire(b, b)

        def body(k, carry):
            for b in range(NBUF):
                j = jnp.minimum(NBUF * k + b, B_PER_W - 1)
                drain(b)
                _extract_row(bufs[b], read_idx(j) & 127, rows_v, j * EMBED_DIM)
                fire(jnp.minimum(j + NBUF, B_PER_W - 1), b)
            return carry

        lax.fori_loop(0, (B_PER_W + NBUF - 1) // NBUF, body, 0)
        for b in range(NBUF):
            drain(b)
        pltpu.sync_copy(rows_v, out_hbm.at[pl.ds(base * EMBED_DIM, B_PER_W * EMBED_DIM)])

    return gather(table_t, class_index)


def _mlp_body(emb_ref, w1_ref, b1_ref, w2_ref, b2_ref, out_ref):
    x = emb_ref[...]
    h = jnp.dot(x, w1_ref[...], preferred_element_type=jnp.float32) + b1_ref[...]
    h = h * jax.nn.sigmoid(h)
    out = jnp.dot(h, w2_ref[...], preferred_element_type=jnp.float32) + b2_ref[...]
    out_ref[...] = out.T


def _tc_mlp(emb, W1, b1, W2, b2):
    blk = 2048
    grid = (BATCH // blk,)
    return pl.pallas_call(
        _mlp_body,
        grid=grid,
        in_specs=[
            pl.BlockSpec((blk, EMBED_DIM), lambda i: (i, 0)),
            pl.BlockSpec((EMBED_DIM, EMBED_DIM), lambda i: (0, 0)),
            pl.BlockSpec((1, EMBED_DIM), lambda i: (0, 0)),
            pl.BlockSpec((EMBED_DIM, EMBED_DIM), lambda i: (0, 0)),
            pl.BlockSpec((1, EMBED_DIM), lambda i: (0, 0)),
        ],
        out_specs=pl.BlockSpec((EMBED_DIM, blk), lambda i: (0, i)),
        out_shape=jax.ShapeDtypeStruct((EMBED_DIM, BATCH), jnp.float32),
        compiler_params=pltpu.CompilerParams(
            dimension_semantics=("arbitrary",),
        ),
    )(emb, W1, b1.reshape(1, EMBED_DIM), W2, b2.reshape(1, EMBED_DIM))


def kernel(class_index, table, W1, b1, W2, b2):
    emb_flat = _sc_gather(table.T, class_index.astype(jnp.int32))
    emb = emb_flat.reshape(BATCH, EMBED_DIM)
    return _tc_mlp(emb, W1, b1, W2, b2).T
